# baseline (speedup 1.0000x reference)
import functools

import jax
import jax.numpy as jnp
from jax.experimental import pallas as pl
from jax.experimental.pallas import tpu as pltpu

_LANE_TILE = 43690


def _round_up(n: int, m: int) -> int:
    return pl.cdiv(n, m) * m


def _mlp_kernel(xt_ref, w1t_ref, w2t_ref, b1_ref, b2_ref, o_ref):
    xt = xt_ref[...].astype(jnp.bfloat16)
    h = jax.lax.dot_general(w1t_ref[...].astype(jnp.bfloat16), xt,
                            (((1,), (0,)), ((), ())),
                            preferred_element_type=jnp.float32)
    h = jnp.maximum(h + jnp.transpose(b1_ref[...]), 0.0)
    y = jax.lax.dot_general(w2t_ref[...].astype(jnp.bfloat16),
                            h.astype(jnp.bfloat16),
                            (((1,), (0,)), ((), ())),
                            preferred_element_type=jnp.float32)
    o_ref[...] = y + jnp.transpose(b2_ref[...])


def kernel(x, w1, b1, w2, b2):
    B, In = x.shape
    H = w1.shape[1]
    Out = w2.shape[1]

    xt = x.T
    w1t = w1.T
    w2t = w2.T

    n_tiles = max(2, pl.cdiv(B, _LANE_TILE))
    n_tiles += n_tiles % 2
    tb = _round_up(pl.cdiv(B, n_tiles), 128)
    grid = (pl.cdiv(B, tb),)

    yt = pl.pallas_call(
        _mlp_kernel,
        out_shape=jax.ShapeDtypeStruct((Out, B), jnp.float32),
        grid=grid,
        in_specs=[
            pl.BlockSpec((In, tb), lambda i: (0, i)),
            pl.BlockSpec((H, In), lambda i: (0, 0)),
            pl.BlockSpec((Out, H), lambda i: (0, 0)),
            pl.BlockSpec((1, H), lambda i: (0, 0)),
            pl.BlockSpec((1, Out), lambda i: (0, 0)),
        ],
        out_specs=pl.BlockSpec((Out, tb), lambda i: (0, i)),
        compiler_params=pltpu.CompilerParams(
            dimension_semantics=("parallel",),
        ),
    )(xt, w1t, w2t, b1, b2)
    return yt.T


# speedup vs baseline: 1.0020x; 1.0020x over previous
import jax
import jax.numpy as jnp
from jax.experimental import pallas as pl
from jax.experimental.pallas import tpu as pltpu

_LANE_TILE = 32768


def _round_up(n: int, m: int) -> int:
    return pl.cdiv(n, m) * m


def _mlp_kernel(xt_ref, w1t_ref, w2t_ref, b1_ref, b2_ref, o_ref):
    xt = xt_ref[...].astype(jnp.bfloat16)
    h = jax.lax.dot_general(w1t_ref[...].astype(jnp.bfloat16), xt,
                            (((1,), (0,)), ((), ())),
                            preferred_element_type=jnp.float32)
    h = jnp.maximum(h + jnp.transpose(b1_ref[...]), 0.0)
    y = jax.lax.dot_general(w2t_ref[...].astype(jnp.bfloat16),
                            h.astype(jnp.bfloat16),
                            (((1,), (0,)), ((), ())),
                            preferred_element_type=jnp.float32)
    o_ref[...] = y + jnp.transpose(b2_ref[...])


def kernel(x, w1, b1, w2, b2):
    B, In = x.shape
    H = w1.shape[1]
    Out = w2.shape[1]

    xt = x.T
    w1t = w1.T
    w2t = w2.T

    n_tiles = max(2, pl.cdiv(B, _LANE_TILE))
    n_tiles += n_tiles % 2
    tb = _round_up(pl.cdiv(B, n_tiles), 128)
    grid = (pl.cdiv(B, tb),)

    yt = pl.pallas_call(
        _mlp_kernel,
        out_shape=jax.ShapeDtypeStruct((Out, B), jnp.float32),
        grid=grid,
        in_specs=[
            pl.BlockSpec((In, tb), lambda i: (0, i)),
            pl.BlockSpec((H, In), lambda i: (0, 0)),
            pl.BlockSpec((Out, H), lambda i: (0, 0)),
            pl.BlockSpec((1, H), lambda i: (0, 0)),
            pl.BlockSpec((1, Out), lambda i: (0, 0)),
        ],
        out_specs=pl.BlockSpec((Out, tb), lambda i: (0, i)),
        compiler_params=pltpu.CompilerParams(
            dimension_semantics=("parallel",),
        ),
    )(xt, w1t, w2t, b1, b2)
    return yt.T
